# baseline (speedup 1.0000x reference)
import functools
import jax
import jax.numpy as jnp
from jax.experimental import pallas as pl
from jax.experimental.pallas import tpu as pltpu

D = 1024
E = 8
H = 512
TM = 256
LANES = 128


def _silu(x):
    return x / (1.0 + jnp.exp(-x))


def _moe_dense_body(x_ref, w1_ref, b1_ref, w2_ref, b2_ref, rw_ref, eb_ref,
                    out_ref, w_scr):
    e = pl.program_id(1)

    @pl.when(e == 0)
    def _compute_router():
        x = x_ref[...]
        logits = jnp.dot(x, rw_ref[...], preferred_element_type=jnp.float32)
        logits = logits + eb_ref[...]
        sp = jnp.maximum(logits, 0.0) + jnp.log(1.0 + jnp.exp(-jnp.abs(logits)))
        act = jnp.sqrt(sp)
        lane = jax.lax.broadcasted_iota(jnp.int32, (TM, LANES), 1)
        actm = jnp.where(lane < E, act, -1.0)
        m1 = jnp.max(actm, axis=1, keepdims=True)
        l1 = jnp.min(jnp.where(actm == m1, lane, LANES), axis=1, keepdims=True)
        oh0 = lane == l1
        act2 = jnp.where(oh0, -1.0, actm)
        m2 = jnp.max(act2, axis=1, keepdims=True)
        l2 = jnp.min(jnp.where(act2 == m2, lane, LANES), axis=1, keepdims=True)
        oh1 = lane == l2
        w = jnp.where(oh0, m1, 0.0) + jnp.where(oh1, m2, 0.0)
        w = jnp.where(lane == E, 1.0, w)
        w_scr[...] = w

    x = x_ref[...]
    h = jnp.dot(x, w1_ref[0], preferred_element_type=jnp.float32) + b1_ref[0]
    h = _silu(h)
    y = jnp.dot(h, w2_ref[0], preferred_element_type=jnp.float32) + b2_ref[0]
    lane_col = jax.lax.broadcasted_iota(jnp.int32, (LANES, 1), 0)
    ohe = (lane_col == pl.program_id(1)).astype(jnp.float32)
    we = jnp.dot(w_scr[...], ohe, preferred_element_type=jnp.float32)
    contrib = y * we

    @pl.when(e == 0)
    def _init():
        out_ref[...] = contrib

    @pl.when(e > 0)
    def _acc():
        out_ref[...] += contrib


def kernel(X, shared_W1, shared_b1, shared_W2, shared_b2,
           routed_W1, routed_b1, routed_W2, routed_b2,
           routing_W, expert_bias):
    B, T, _ = X.shape
    N = B * T
    nt = N // TM
    x2 = X.reshape(N, D)

    W1s = jnp.concatenate([routed_W1, shared_W1[None]], axis=0)
    W2s = jnp.concatenate([routed_W2, shared_W2[None]], axis=0)
    b1s = jnp.concatenate([routed_b1, shared_b1[None]], axis=0).reshape(E + 1, 1, H)
    b2s = jnp.concatenate([routed_b2, shared_b2[None]], axis=0).reshape(E + 1, 1, D)
    rw_pad = jnp.pad(routing_W, ((0, 0), (0, LANES - E)))
    eb_pad = jnp.pad(expert_bias, (0, LANES - E)).reshape(1, LANES)

    out = pl.pallas_call(
        _moe_dense_body,
        grid=(nt, E + 1),
        in_specs=[
            pl.BlockSpec((TM, D), lambda t, e: (t, 0)),
            pl.BlockSpec((1, D, H), lambda t, e: (e, 0, 0)),
            pl.BlockSpec((1, 1, H), lambda t, e: (e, 0, 0)),
            pl.BlockSpec((1, H, D), lambda t, e: (e, 0, 0)),
            pl.BlockSpec((1, 1, D), lambda t, e: (e, 0, 0)),
            pl.BlockSpec((D, LANES), lambda t, e: (0, 0)),
            pl.BlockSpec((1, LANES), lambda t, e: (0, 0)),
        ],
        out_specs=pl.BlockSpec((TM, D), lambda t, e: (t, 0)),
        out_shape=jax.ShapeDtypeStruct((N, D), jnp.float32),
        scratch_shapes=[pltpu.VMEM((TM, LANES), jnp.float32)],
        compiler_params=pltpu.CompilerParams(
            dimension_semantics=("parallel", "arbitrary"),
        ),
    )(x2, W1s, b1s, W2s, b2s, rw_pad, eb_pad)
    return out.reshape(B, T, D)
